# baseline (speedup 1.0000x reference)
import functools

import jax
import jax.numpy as jnp
from jax import lax
from jax.experimental import pallas as pl
from jax.experimental.pallas import tpu as pltpu
from jax.experimental.pallas import tpu_sc as plsc

N = 5000
E = 160000
SEQ = 128
DEMB = SEQ + 1

NC = 2
NS = 16
LANES = 16
NW = NC * NS

DP = 128
NP = 5120
CH = 128
EPT = 5120
EP = NW * EPT
NCHUNK = EPT // CH
RPT = NP // NS

_f32 = jnp.float32
_i32 = jnp.int32

_SC_MESH = plsc.VectorSubcoreMesh(
    core_axis_name="c", subcore_axis_name="s", num_cores=NC, num_subcores=NS
)
_SC_PARAMS = pltpu.CompilerParams(needs_layout_passes=False,
                                  use_tc_tiling_on_sc=False)


def _deg_body(col_hbm, w_hbm, deg_out, col_v, w_v, deg_v):
    c = lax.axis_index("c")
    s = lax.axis_index("s")
    wid = c * NS + s

    def zero(i, carry):
        deg_v[pl.ds(i * LANES, LANES)] = jnp.zeros((LANES,), _f32)
        return carry

    lax.fori_loop(0, NP // LANES, zero, 0)

    base = wid * EPT
    pltpu.sync_copy(col_hbm.at[pl.ds(base, EPT)], col_v)
    pltpu.sync_copy(w_hbm.at[pl.ds(base, EPT)], w_v)

    def body(i, carry):
        idx = col_v[pl.ds(i * LANES, LANES)]
        vals = w_v[pl.ds(i * LANES, LANES)]
        plsc.addupdate_scatter(deg_v, [idx], vals)
        return carry

    lax.fori_loop(0, EPT // LANES, body, 0)
    pltpu.sync_copy(deg_v, deg_out.at[wid])


_deg_call = functools.partial(
    pl.kernel,
    out_type=jax.ShapeDtypeStruct((NW, NP), _f32),
    mesh=_SC_MESH,
    scratch_types=[
        pltpu.VMEM((EPT,), _i32),
        pltpu.VMEM((EPT,), _f32),
        pltpu.VMEM((NP,), _f32),
    ],
    compiler_params=_SC_PARAMS,
)(_deg_body)


DH = DP // NC
ES = EP // NS
NCH = ES // CH
NBUF = 5
NGRP = NCH // NBUF


def _agg_body(row_hbm, col_hbm, w_hbm, x_hbm, xl_hbm, dinv_hbm,
              out_hbm, outl_hbm,
              ridx, cidx, wv, dinv_v, xl_v, accl_v, rows, acc_sh, x_sh,
              *sems):
    gsem = sems[:NBUF]
    ssem = sems[NBUF:]
    c = lax.axis_index("c")
    s = lax.axis_index("s")

    abase = s * RPT
    pltpu.sync_copy(x_hbm.at[c, pl.ds(abase, RPT)], x_sh.at[pl.ds(abase, RPT)])
    pltpu.sync_copy(dinv_hbm, dinv_v)
    pltpu.sync_copy(xl_hbm, xl_v)
    pltpu.sync_copy(row_hbm.at[s], ridx)
    pltpu.sync_copy(col_hbm.at[s], cidx)
    pltpu.sync_copy(w_hbm.at[s], wv)

    def zlast(i, carry):
        accl_v[pl.ds(i * LANES, LANES)] = jnp.zeros((LANES,), _f32)
        return carry

    lax.fori_loop(0, NP // LANES, zlast, 0)

    def psg(g, carry):
        def psi(i, carry2):
            sl = pl.ds(i * LANES, LANES)
            ri = ridx[g, sl]
            wsc = wv[g, sl] * plsc.load_gather(dinv_v, [ri])
            wv[g, sl] = wsc
            xl = plsc.load_gather(xl_v, [ri])
            plsc.addupdate_scatter(accl_v, [cidx[g, sl]], wsc * xl)
            return carry2
        lax.fori_loop(0, CH // LANES, psi, 0)
        return carry

    lax.fori_loop(0, NCH, psg, 0)

    @pl.when(c == 0)
    def _():
        pltpu.sync_copy(accl_v, outl_hbm.at[s])

    def zrow(i, carry):
        for f in range(DH // LANES):
            rows[0, i, pl.ds(f * LANES, LANES)] = jnp.zeros((LANES,), _f32)
        return carry

    lax.fori_loop(0, CH, zrow, 0)
    pltpu.sync_copy(rows.at[0], acc_sh.at[pl.ds(abase, CH)])
    pltpu.sync_copy(rows.at[0], acc_sh.at[pl.ds(abase + CH, CH)])
    pltpu.sync_copy(rows.at[0, pl.ds(0, RPT - 2 * CH)],
                    acc_sh.at[pl.ds(abase + 2 * CH, RPT - 2 * CH)])
    plsc.subcore_barrier()

    def start_gather(g, b):
        pltpu.async_copy(x_sh.at[ridx.at[g]], rows.at[b], gsem[b])

    def wait_gather(g, b):
        pltpu.make_async_copy(x_sh.at[ridx.at[g]], rows.at[b], gsem[b]).wait()

    def start_scatter(g, b):
        pltpu.async_copy(rows.at[b], acc_sh.at[cidx.at[g]], ssem[b], add=True)

    def wait_scatter(g, b):
        pltpu.make_async_copy(rows.at[b], acc_sh.at[cidx.at[g]],
                              ssem[b]).wait()

    for b in range(NBUF - 1):
        start_gather(b, b)

    def group(grp, carry):
        for b in range(NBUF):
            g = grp * NBUF + b
            bp = (b - 1) % NBUF
            wait_gather(g, b)
            g16 = lax.broadcast(g, (LANES,))

            def edge(j, carry2):
                wvec = plsc.load_gather(wv, [g16, lax.broadcast(j, (LANES,))])
                for f in range(DH // LANES):
                    sl = pl.ds(f * LANES, LANES)
                    rows[b, j, sl] = rows[b, j, sl] * wvec
                return carry2

            lax.fori_loop(0, CH, edge, 0, unroll=2)
            start_scatter(g, b)
            if b == 0:
                @pl.when(grp > 0)
                def _():
                    wait_scatter(g - 1, bp)
                start_gather(g + NBUF - 1, bp)
            else:
                wait_scatter(g - 1, bp)

                @pl.when(grp < NGRP - 1)
                def _():
                    start_gather(g + NBUF - 1, bp)
        return carry

    lax.fori_loop(0, NGRP, group, 0)
    wait_scatter(NCH - 1, (NCH - 1) % NBUF)
    plsc.subcore_barrier()
    pltpu.sync_copy(acc_sh.at[pl.ds(abase, RPT)],
                    out_hbm.at[c, pl.ds(abase, RPT)])


_agg_call = functools.partial(
    pl.kernel,
    out_type=(jax.ShapeDtypeStruct((NC, NP, DH), _f32),
              jax.ShapeDtypeStruct((NS, NP), _f32)),
    mesh=_SC_MESH,
    scratch_types=[
        pltpu.VMEM((NCH, CH), _i32),
        pltpu.VMEM((NCH, CH), _i32),
        pltpu.VMEM((NCH, CH), _f32),
        pltpu.VMEM((NP,), _f32),
        pltpu.VMEM((NP,), _f32),
        pltpu.VMEM((NP,), _f32),
        pltpu.VMEM((NBUF, CH, DH), _f32),
        pltpu.VMEM_SHARED((NP, DH), _f32),
        pltpu.VMEM_SHARED((NP, DH), _f32),
    ] + [pltpu.SemaphoreType.DMA] * (2 * NBUF),
    compiler_params=_SC_PARAMS,
)(_agg_body)


def _layer_body(p_ref, accl_ref, embm_ref, embl_ref, dinv_ref,
                wmm_ref, wml_ref, wlm_ref, wll_ref, bm_ref, bl_ref,
                outm_ref, outl_ref, *, relu):
    dv = dinv_ref[...]
    acc = jnp.concatenate([p_ref[0], p_ref[1]], axis=1)
    embm = jnp.concatenate([embm_ref[0], embm_ref[1]], axis=1)
    aggm = dv * acc + (dv * dv) * embm
    aggl = dv * accl_ref[...] + (dv * dv) * embl_ref[...]
    ym = (jnp.dot(aggm, wmm_ref[...], preferred_element_type=_f32)
          + aggl * wlm_ref[...] + bm_ref[...])
    yl = (jnp.dot(aggm, wml_ref[...], preferred_element_type=_f32)
          + aggl * wll_ref[...] + bl_ref[...])
    if relu:
        ym = jnp.maximum(ym, 0.0)
        yl = jnp.maximum(yl, 0.0)
    outm_ref[0] = ym[:, :DH]
    outm_ref[1] = ym[:, DH:]
    outl_ref[...] = yl


def _layer_call(parts, accl, embm, embl, dinv2d, wmm, wml, wlm, wll, bm, bl,
                relu):
    return pl.pallas_call(
        functools.partial(_layer_body, relu=relu),
        out_shape=(jax.ShapeDtypeStruct((NC, NP, DH), _f32),
                   jax.ShapeDtypeStruct((NP, 1), _f32)),
    )(parts, accl, embm, embl, dinv2d, wmm, wml, wlm, wll, bm, bl)


def _final_body(embm_ref, embl_ref, pidx_ref,
                wqmm_ref, wqml_ref, wqlm_ref, wqll_ref, bqm_ref, bql_ref,
                wkmm_ref, wkml_ref, wklm_ref, wkll_ref, bkm_ref, bkl_ref,
                d0wm_ref, d0wl_ref, d0b_ref, d1w_ref, d1b_ref,
                d2w_ref, d2b_ref, ow_ref, ob_ref, out_ref):
    embm = jnp.concatenate([embm_ref[0], embm_ref[1]], axis=1)
    embl = embl_ref[...]
    p = pidx_ref[0]
    rid = lax.broadcasted_iota(_i32, (NP, 1), 0)
    qsel = (rid == p).astype(_f32)
    qrm = jnp.sum(embm * qsel, axis=0, keepdims=True)
    qrl = jnp.sum(embl * qsel, axis=0, keepdims=True)
    qm = (jnp.dot(qrm, wqmm_ref[...], preferred_element_type=_f32)
          + qrl * wqlm_ref[...] + bqm_ref[...])
    ql = (jnp.dot(qrm, wqml_ref[...], preferred_element_type=_f32)
          + qrl * wqll_ref[...] + bql_ref[...])
    km = (jnp.dot(embm, wkmm_ref[...], preferred_element_type=_f32)
          + embl * wklm_ref[...] + bkm_ref[...])
    kl = (jnp.dot(embm, wkml_ref[...], preferred_element_type=_f32)
          + embl * wkll_ref[...] + bkl_ref[...])
    s = jnp.sum(km * qm, axis=1, keepdims=True) + kl * ql
    s = s * (1.0 / (float(DEMB) ** 0.5))
    valid = rid < N
    s = jnp.where(valid, s, -1e30)
    m = jnp.max(s)
    ex = jnp.where(valid, jnp.exp(s - m), 0.0)
    aw = ex / jnp.sum(ex)
    cm = jnp.sum(embm * aw, axis=0, keepdims=True)
    cl = jnp.sum(embl * aw, axis=0, keepdims=True)
    h = (jnp.dot(cm, d0wm_ref[...], preferred_element_type=_f32)
         + cl * d0wl_ref[...] + d0b_ref[...])
    h = jnp.maximum(h, 0.0)
    h = jnp.dot(h, d1w_ref[...], preferred_element_type=_f32) + d1b_ref[...]
    h = jnp.maximum(h, 0.0)
    h = jnp.dot(h, d2w_ref[...], preferred_element_type=_f32) + d2b_ref[...]
    h = jnp.maximum(h, 0.0)
    out_ref[...] = (jnp.dot(h, ow_ref[...], preferred_element_type=_f32)
                    + ob_ref[...])


def _final_call(embm, embl, pidx, *ws):
    in_specs = [pl.BlockSpec(memory_space=pltpu.VMEM),
                pl.BlockSpec(memory_space=pltpu.VMEM),
                pl.BlockSpec(memory_space=pltpu.SMEM)]
    in_specs += [pl.BlockSpec(memory_space=pltpu.VMEM)] * len(ws)
    return pl.pallas_call(
        _final_body,
        out_shape=jax.ShapeDtypeStruct((1, SEQ), _f32),
        in_specs=in_specs,
    )(embm, embl, pidx, *ws)


def _row(v):
    return v.reshape(1, -1).astype(_f32)


def kernel(src, edge_index, edge_weight, pert_idx, gnn_w0, gnn_b0, gnn_w1,
           gnn_b1, attn_in_w, attn_in_b, attn_out_w, attn_out_b, dec_w0,
           dec_b0, dec_w1, dec_b1, dec_w2, dec_b2, out_w, out_b):
    pert_idx = jnp.asarray(pert_idx, _i32)

    pad_e = EP - E
    row_p = jnp.concatenate([edge_index[0], jnp.zeros((pad_e,), _i32)])
    col_p = jnp.concatenate([edge_index[1], jnp.zeros((pad_e,), _i32)])
    w_p = jnp.concatenate([edge_weight, jnp.zeros((pad_e,), _f32)])
    row3 = row_p.reshape(NS, NCH, CH)
    col3 = col_p.reshape(NS, NCH, CH)
    w3 = w_p.reshape(NS, NCH, CH)

    emb0f = jnp.zeros((NP, DP), _f32).at[:N, :SEQ].set(src)
    emb0m = jnp.stack([emb0f[:, :DH], emb0f[:, DH:]])
    emb0l = jnp.zeros((NP, 1), _f32).at[pert_idx, 0].set(1.0)

    def _split(wt):
        return wt[:SEQ], wt[SEQ:SEQ + 1]

    w0mm, w0lm = _split(gnn_w0.T[:, :SEQ])
    w0ml, w0ll = _split(gnn_w0.T[:, SEQ:DEMB])
    w1mm, w1lm = _split(gnn_w1.T[:, :SEQ])
    w1ml, w1ll = _split(gnn_w1.T[:, SEQ:DEMB])
    b0m, b0l = _row(gnn_b0[:SEQ]), _row(gnn_b0[SEQ:DEMB])
    b1m, b1l = _row(gnn_b1[:SEQ]), _row(gnn_b1[SEQ:DEMB])

    wq = attn_in_w[:DEMB].T
    wk = attn_in_w[DEMB:2 * DEMB].T
    wqmm, wqlm = _split(wq[:, :SEQ])
    wqml, wqll = _split(wq[:, SEQ:DEMB])
    wkmm, wklm = _split(wk[:, :SEQ])
    wkml, wkll = _split(wk[:, SEQ:DEMB])
    bqm, bql = _row(attn_in_b[:SEQ]), _row(attn_in_b[SEQ:DEMB])
    bkm = _row(attn_in_b[DEMB:DEMB + SEQ])
    bkl = _row(attn_in_b[DEMB + SEQ:2 * DEMB])

    d0wm, d0wl = _split(dec_w0.T)
    d0b = _row(dec_b0)
    d1w, d1b = dec_w1.T, _row(dec_b1)
    d2w, d2b = dec_w2.T, _row(dec_b2)
    ow, ob = out_w.T, _row(out_b)

    deg_parts = _deg_call(col_p, w_p)
    self_loop = (jnp.arange(NP) < N).astype(_f32)
    deg = jnp.sum(deg_parts, axis=0) + self_loop
    dinv = jnp.where(deg > 0, 1.0 / jnp.sqrt(deg), 0.0)
    dinv2d = dinv.reshape(NP, 1)

    parts0, pl0 = _agg_call(row3, col3, w3, emb0m,
                            emb0l.reshape(NP), dinv)
    accl0 = jnp.sum(pl0, axis=0).reshape(NP, 1)
    emb1m, emb1l = _layer_call(parts0, accl0, emb0m, emb0l, dinv2d,
                               w0mm, w0ml, w0lm, w0ll, b0m, b0l, relu=True)

    parts1, pl1 = _agg_call(row3, col3, w3, emb1m,
                            emb1l.reshape(NP), dinv)
    accl1 = jnp.sum(pl1, axis=0).reshape(NP, 1)
    emb2m, emb2l = _layer_call(parts1, accl1, emb1m, emb1l, dinv2d,
                               w1mm, w1ml, w1lm, w1ll, b1m, b1l, relu=False)

    pidx = pert_idx.reshape(1)
    return _final_call(emb2m, emb2l, pidx,
                       wqmm, wqml, wqlm, wqll, bqm, bql,
                       wkmm, wkml, wklm, wkll, bkm, bkl,
                       d0wm, d0wl, d0b, d1w, d1b, d2w, d2b, ow, ob)


# speedup vs baseline: 1.0083x; 1.0083x over previous
import functools

import jax
import jax.numpy as jnp
from jax import lax
from jax.experimental import pallas as pl
from jax.experimental.pallas import tpu as pltpu
from jax.experimental.pallas import tpu_sc as plsc

N = 5000
E = 160000
SEQ = 128
DEMB = SEQ + 1

NC = 2
NS = 16
LANES = 16
NW = NC * NS

DP = 128
NP = 5120
CH = 128
EPT = 5120
EP = NW * EPT
NCHUNK = EPT // CH
RPT = NP // NS

_f32 = jnp.float32
_i32 = jnp.int32

_SC_MESH = plsc.VectorSubcoreMesh(
    core_axis_name="c", subcore_axis_name="s", num_cores=NC, num_subcores=NS
)
_SC_PARAMS = pltpu.CompilerParams(needs_layout_passes=False,
                                  use_tc_tiling_on_sc=False)


def _deg_body(col_hbm, w_hbm, deg_out, col_v, w_v, deg_v):
    c = lax.axis_index("c")
    s = lax.axis_index("s")
    wid = c * NS + s

    def zero(i, carry):
        deg_v[pl.ds(i * LANES, LANES)] = jnp.zeros((LANES,), _f32)
        return carry

    lax.fori_loop(0, NP // LANES, zero, 0)

    base = wid * EPT
    pltpu.sync_copy(col_hbm.at[pl.ds(base, EPT)], col_v)
    pltpu.sync_copy(w_hbm.at[pl.ds(base, EPT)], w_v)

    def body(i, carry):
        idx = col_v[pl.ds(i * LANES, LANES)]
        vals = w_v[pl.ds(i * LANES, LANES)]
        plsc.addupdate_scatter(deg_v, [idx], vals)
        return carry

    lax.fori_loop(0, EPT // LANES, body, 0)
    pltpu.sync_copy(deg_v, deg_out.at[wid])


_deg_call = functools.partial(
    pl.kernel,
    out_type=jax.ShapeDtypeStruct((NW, NP), _f32),
    mesh=_SC_MESH,
    scratch_types=[
        pltpu.VMEM((EPT,), _i32),
        pltpu.VMEM((EPT,), _f32),
        pltpu.VMEM((NP,), _f32),
    ],
    compiler_params=_SC_PARAMS,
)(_deg_body)


DH = DP // NC
ES = EP // NS
NCH = ES // CH
NBUF = 5
NGRP = NCH // NBUF


def _agg_body(row_hbm, col_hbm, w_hbm, x_hbm, xl_hbm, dinv_hbm,
              out_hbm, outl_hbm,
              ridx, cidx, wv, dinv_v, xl_v, accl_v, rows, acc_sh, x_sh,
              *sems):
    gsem = sems[:NBUF]
    ssem = sems[NBUF:]
    c = lax.axis_index("c")
    s = lax.axis_index("s")

    abase = s * RPT
    pltpu.sync_copy(x_hbm.at[c, pl.ds(abase, RPT)], x_sh.at[pl.ds(abase, RPT)])
    pltpu.sync_copy(dinv_hbm, dinv_v)
    pltpu.sync_copy(xl_hbm, xl_v)
    pltpu.sync_copy(row_hbm.at[s], ridx)
    pltpu.sync_copy(col_hbm.at[s], cidx)
    pltpu.sync_copy(w_hbm.at[s], wv)

    def zlast(i, carry):
        accl_v[pl.ds(i * LANES, LANES)] = jnp.zeros((LANES,), _f32)
        return carry

    lax.fori_loop(0, NP // LANES, zlast, 0)

    def psg(g, carry):
        def psi(i, carry2):
            sl = pl.ds(i * LANES, LANES)
            ri = ridx[g, sl]
            wsc = wv[g, sl] * plsc.load_gather(dinv_v, [ri])
            wv[g, sl] = wsc
            xl = plsc.load_gather(xl_v, [ri])
            plsc.addupdate_scatter(accl_v, [cidx[g, sl]], wsc * xl)
            return carry2
        lax.fori_loop(0, CH // LANES, psi, 0)
        return carry

    lax.fori_loop(0, NCH, psg, 0)

    @pl.when(c == 0)
    def _():
        pltpu.sync_copy(accl_v, outl_hbm.at[s])

    def zrow(i, carry):
        for f in range(DH // LANES):
            rows[0, i, pl.ds(f * LANES, LANES)] = jnp.zeros((LANES,), _f32)
        return carry

    lax.fori_loop(0, CH, zrow, 0)
    pltpu.sync_copy(rows.at[0], acc_sh.at[pl.ds(abase, CH)])
    pltpu.sync_copy(rows.at[0], acc_sh.at[pl.ds(abase + CH, CH)])
    pltpu.sync_copy(rows.at[0, pl.ds(0, RPT - 2 * CH)],
                    acc_sh.at[pl.ds(abase + 2 * CH, RPT - 2 * CH)])
    plsc.subcore_barrier()

    def start_gather(g, b):
        pltpu.async_copy(x_sh.at[ridx.at[g]], rows.at[b], gsem[b])

    def wait_gather(g, b):
        pltpu.make_async_copy(x_sh.at[ridx.at[g]], rows.at[b], gsem[b]).wait()

    def start_scatter(g, b):
        pltpu.async_copy(rows.at[b], acc_sh.at[cidx.at[g]], ssem[b], add=True)

    def wait_scatter(g, b):
        pltpu.make_async_copy(rows.at[b], acc_sh.at[cidx.at[g]],
                              ssem[b]).wait()

    for b in range(NBUF - 1):
        start_gather(b, b)

    def group(grp, carry):
        for b in range(NBUF):
            g = grp * NBUF + b
            bp = (b - 1) % NBUF
            wait_gather(g, b)
            g16 = lax.broadcast(g, (LANES,))

            def edge(j, carry2):
                wvec = plsc.load_gather(wv, [g16, lax.broadcast(j, (LANES,))])
                for f in range(DH // LANES):
                    sl = pl.ds(f * LANES, LANES)
                    rows[b, j, sl] = rows[b, j, sl] * wvec
                return carry2

            lax.fori_loop(0, CH, edge, 0, unroll=4)
            start_scatter(g, b)
            if b == 0:
                @pl.when(grp > 0)
                def _():
                    wait_scatter(g - 1, bp)
                start_gather(g + NBUF - 1, bp)
            else:
                wait_scatter(g - 1, bp)

                @pl.when(grp < NGRP - 1)
                def _():
                    start_gather(g + NBUF - 1, bp)
        return carry

    lax.fori_loop(0, NGRP, group, 0)
    wait_scatter(NCH - 1, (NCH - 1) % NBUF)
    plsc.subcore_barrier()
    pltpu.sync_copy(acc_sh.at[pl.ds(abase, RPT)],
                    out_hbm.at[c, pl.ds(abase, RPT)])


_agg_call = functools.partial(
    pl.kernel,
    out_type=(jax.ShapeDtypeStruct((NC, NP, DH), _f32),
              jax.ShapeDtypeStruct((NS, NP), _f32)),
    mesh=_SC_MESH,
    scratch_types=[
        pltpu.VMEM((NCH, CH), _i32),
        pltpu.VMEM((NCH, CH), _i32),
        pltpu.VMEM((NCH, CH), _f32),
        pltpu.VMEM((NP,), _f32),
        pltpu.VMEM((NP,), _f32),
        pltpu.VMEM((NP,), _f32),
        pltpu.VMEM((NBUF, CH, DH), _f32),
        pltpu.VMEM_SHARED((NP, DH), _f32),
        pltpu.VMEM_SHARED((NP, DH), _f32),
    ] + [pltpu.SemaphoreType.DMA] * (2 * NBUF),
    compiler_params=_SC_PARAMS,
)(_agg_body)


def _layer_body(p_ref, accl_ref, embm_ref, embl_ref, dinv_ref,
                wmm_ref, wml_ref, wlm_ref, wll_ref, bm_ref, bl_ref,
                outm_ref, outl_ref, *, relu):
    dv = dinv_ref[...]
    acc = jnp.concatenate([p_ref[0], p_ref[1]], axis=1)
    embm = jnp.concatenate([embm_ref[0], embm_ref[1]], axis=1)
    aggm = dv * acc + (dv * dv) * embm
    aggl = dv * accl_ref[...] + (dv * dv) * embl_ref[...]
    ym = (jnp.dot(aggm, wmm_ref[...], preferred_element_type=_f32)
          + aggl * wlm_ref[...] + bm_ref[...])
    yl = (jnp.dot(aggm, wml_ref[...], preferred_element_type=_f32)
          + aggl * wll_ref[...] + bl_ref[...])
    if relu:
        ym = jnp.maximum(ym, 0.0)
        yl = jnp.maximum(yl, 0.0)
    outm_ref[0] = ym[:, :DH]
    outm_ref[1] = ym[:, DH:]
    outl_ref[...] = yl


def _layer_call(parts, accl, embm, embl, dinv2d, wmm, wml, wlm, wll, bm, bl,
                relu):
    return pl.pallas_call(
        functools.partial(_layer_body, relu=relu),
        out_shape=(jax.ShapeDtypeStruct((NC, NP, DH), _f32),
                   jax.ShapeDtypeStruct((NP, 1), _f32)),
    )(parts, accl, embm, embl, dinv2d, wmm, wml, wlm, wll, bm, bl)


def _final_body(embm_ref, embl_ref, pidx_ref,
                wqmm_ref, wqml_ref, wqlm_ref, wqll_ref, bqm_ref, bql_ref,
                wkmm_ref, wkml_ref, wklm_ref, wkll_ref, bkm_ref, bkl_ref,
                d0wm_ref, d0wl_ref, d0b_ref, d1w_ref, d1b_ref,
                d2w_ref, d2b_ref, ow_ref, ob_ref, out_ref):
    embm = jnp.concatenate([embm_ref[0], embm_ref[1]], axis=1)
    embl = embl_ref[...]
    p = pidx_ref[0]
    rid = lax.broadcasted_iota(_i32, (NP, 1), 0)
    qsel = (rid == p).astype(_f32)
    qrm = jnp.sum(embm * qsel, axis=0, keepdims=True)
    qrl = jnp.sum(embl * qsel, axis=0, keepdims=True)
    qm = (jnp.dot(qrm, wqmm_ref[...], preferred_element_type=_f32)
          + qrl * wqlm_ref[...] + bqm_ref[...])
    ql = (jnp.dot(qrm, wqml_ref[...], preferred_element_type=_f32)
          + qrl * wqll_ref[...] + bql_ref[...])
    km = (jnp.dot(embm, wkmm_ref[...], preferred_element_type=_f32)
          + embl * wklm_ref[...] + bkm_ref[...])
    kl = (jnp.dot(embm, wkml_ref[...], preferred_element_type=_f32)
          + embl * wkll_ref[...] + bkl_ref[...])
    s = jnp.sum(km * qm, axis=1, keepdims=True) + kl * ql
    s = s * (1.0 / (float(DEMB) ** 0.5))
    valid = rid < N
    s = jnp.where(valid, s, -1e30)
    m = jnp.max(s)
    ex = jnp.where(valid, jnp.exp(s - m), 0.0)
    aw = ex / jnp.sum(ex)
    cm = jnp.sum(embm * aw, axis=0, keepdims=True)
    cl = jnp.sum(embl * aw, axis=0, keepdims=True)
    h = (jnp.dot(cm, d0wm_ref[...], preferred_element_type=_f32)
         + cl * d0wl_ref[...] + d0b_ref[...])
    h = jnp.maximum(h, 0.0)
    h = jnp.dot(h, d1w_ref[...], preferred_element_type=_f32) + d1b_ref[...]
    h = jnp.maximum(h, 0.0)
    h = jnp.dot(h, d2w_ref[...], preferred_element_type=_f32) + d2b_ref[...]
    h = jnp.maximum(h, 0.0)
    out_ref[...] = (jnp.dot(h, ow_ref[...], preferred_element_type=_f32)
                    + ob_ref[...])


def _final_call(embm, embl, pidx, *ws):
    in_specs = [pl.BlockSpec(memory_space=pltpu.VMEM),
                pl.BlockSpec(memory_space=pltpu.VMEM),
                pl.BlockSpec(memory_space=pltpu.SMEM)]
    in_specs += [pl.BlockSpec(memory_space=pltpu.VMEM)] * len(ws)
    return pl.pallas_call(
        _final_body,
        out_shape=jax.ShapeDtypeStruct((1, SEQ), _f32),
        in_specs=in_specs,
    )(embm, embl, pidx, *ws)


def _row(v):
    return v.reshape(1, -1).astype(_f32)


def kernel(src, edge_index, edge_weight, pert_idx, gnn_w0, gnn_b0, gnn_w1,
           gnn_b1, attn_in_w, attn_in_b, attn_out_w, attn_out_b, dec_w0,
           dec_b0, dec_w1, dec_b1, dec_w2, dec_b2, out_w, out_b):
    pert_idx = jnp.asarray(pert_idx, _i32)

    pad_e = EP - E
    row_p = jnp.concatenate([edge_index[0], jnp.zeros((pad_e,), _i32)])
    col_p = jnp.concatenate([edge_index[1], jnp.zeros((pad_e,), _i32)])
    w_p = jnp.concatenate([edge_weight, jnp.zeros((pad_e,), _f32)])
    row3 = row_p.reshape(NS, NCH, CH)
    col3 = col_p.reshape(NS, NCH, CH)
    w3 = w_p.reshape(NS, NCH, CH)

    emb0f = jnp.zeros((NP, DP), _f32).at[:N, :SEQ].set(src)
    emb0m = jnp.stack([emb0f[:, :DH], emb0f[:, DH:]])
    emb0l = jnp.zeros((NP, 1), _f32).at[pert_idx, 0].set(1.0)

    def _split(wt):
        return wt[:SEQ], wt[SEQ:SEQ + 1]

    w0mm, w0lm = _split(gnn_w0.T[:, :SEQ])
    w0ml, w0ll = _split(gnn_w0.T[:, SEQ:DEMB])
    w1mm, w1lm = _split(gnn_w1.T[:, :SEQ])
    w1ml, w1ll = _split(gnn_w1.T[:, SEQ:DEMB])
    b0m, b0l = _row(gnn_b0[:SEQ]), _row(gnn_b0[SEQ:DEMB])
    b1m, b1l = _row(gnn_b1[:SEQ]), _row(gnn_b1[SEQ:DEMB])

    wq = attn_in_w[:DEMB].T
    wk = attn_in_w[DEMB:2 * DEMB].T
    wqmm, wqlm = _split(wq[:, :SEQ])
    wqml, wqll = _split(wq[:, SEQ:DEMB])
    wkmm, wklm = _split(wk[:, :SEQ])
    wkml, wkll = _split(wk[:, SEQ:DEMB])
    bqm, bql = _row(attn_in_b[:SEQ]), _row(attn_in_b[SEQ:DEMB])
    bkm = _row(attn_in_b[DEMB:DEMB + SEQ])
    bkl = _row(attn_in_b[DEMB + SEQ:2 * DEMB])

    d0wm, d0wl = _split(dec_w0.T)
    d0b = _row(dec_b0)
    d1w, d1b = dec_w1.T, _row(dec_b1)
    d2w, d2b = dec_w2.T, _row(dec_b2)
    ow, ob = out_w.T, _row(out_b)

    deg_parts = _deg_call(col_p, w_p)
    self_loop = (jnp.arange(NP) < N).astype(_f32)
    deg = jnp.sum(deg_parts, axis=0) + self_loop
    dinv = jnp.where(deg > 0, 1.0 / jnp.sqrt(deg), 0.0)
    dinv2d = dinv.reshape(NP, 1)

    parts0, pl0 = _agg_call(row3, col3, w3, emb0m,
                            emb0l.reshape(NP), dinv)
    accl0 = jnp.sum(pl0, axis=0).reshape(NP, 1)
    emb1m, emb1l = _layer_call(parts0, accl0, emb0m, emb0l, dinv2d,
                               w0mm, w0ml, w0lm, w0ll, b0m, b0l, relu=True)

    parts1, pl1 = _agg_call(row3, col3, w3, emb1m,
                            emb1l.reshape(NP), dinv)
    accl1 = jnp.sum(pl1, axis=0).reshape(NP, 1)
    emb2m, emb2l = _layer_call(parts1, accl1, emb1m, emb1l, dinv2d,
                               w1mm, w1ml, w1lm, w1ll, b1m, b1l, relu=False)

    pidx = pert_idx.reshape(1)
    return _final_call(emb2m, emb2l, pidx,
                       wqmm, wqml, wqlm, wqll, bqm, bql,
                       wkmm, wkml, wklm, wkll, bkm, bkl,
                       d0wm, d0wl, d0b, d1w, d1b, d2w, d2b, ow, ob)
